# Initial kernel scaffold; baseline (speedup 1.0000x reference)
import jax
import jax.numpy as jnp
from jax.experimental import pallas as pl


def kernel(points, lengths, W1, b1, W2, b2, W3, b3, W4, b4):
    raise NotImplementedError("write your pallas kernel here")


# speedup vs baseline: 13.9357x; 13.9357x over previous
import jax
import jax.numpy as jnp
from jax.experimental import pallas as pl
from jax.experimental.pallas import tpu as pltpu

B, N, C = 8, 2048, 3
G = 128
K = 32
TOKEN_DIM = 384
NUM_MASKED = int(G * 0.6)
LU = G - NUM_MASKED
GB = 32


def _fps_kernel(p_ref, len_ref, c_ref):
    x = p_ref[:, 0, :]
    y = p_ref[:, 1, :]
    z = p_ref[:, 2, :]
    lens = len_ref[:, 0, :]
    iota_n = jax.lax.broadcasted_iota(jnp.int32, (B, N), 1)
    valid = iota_n < lens
    iota_g = jax.lax.broadcasted_iota(jnp.int32, (B, G), 1)
    inf = jnp.inf

    dist0 = jnp.where(valid, inf, -inf)
    zeros_g = jnp.zeros((B, G), jnp.float32)
    last0 = jnp.zeros((B, 1), jnp.int32)

    def body(i, carry):
        dist, cxs, cys, czs, last = carry
        ohl = (iota_n == last).astype(jnp.float32)
        cx = jnp.sum(x * ohl, axis=1, keepdims=True)
        cy = jnp.sum(y * ohl, axis=1, keepdims=True)
        cz = jnp.sum(z * ohl, axis=1, keepdims=True)
        upd = iota_g == (i - 1)
        cxs = jnp.where(upd, cx, cxs)
        cys = jnp.where(upd, cy, cys)
        czs = jnp.where(upd, cz, czs)
        dx = x - cx
        dy = y - cy
        dz = z - cz
        d = dx * dx + dy * dy + dz * dz
        dist = jnp.where(valid, jnp.minimum(dist, d), -inf)
        m = jnp.max(dist, axis=1, keepdims=True)
        nxt = jnp.min(jnp.where(dist == m, iota_n, N), axis=1, keepdims=True)
        return dist, cxs, cys, czs, nxt

    _, cxs, cys, czs, last = jax.lax.fori_loop(
        1, G, body, (dist0, zeros_g, zeros_g, zeros_g, last0))
    ohl = (iota_n == last).astype(jnp.float32)
    upd = iota_g == (G - 1)
    cxs = jnp.where(upd, jnp.sum(x * ohl, axis=1, keepdims=True), cxs)
    cys = jnp.where(upd, jnp.sum(y * ohl, axis=1, keepdims=True), cys)
    czs = jnp.where(upd, jnp.sum(z * ohl, axis=1, keepdims=True), czs)
    c_ref[:, 0, :] = cxs
    c_ref[:, 1, :] = cys
    c_ref[:, 2, :] = czs


def _knn_kernel(p13_ref, c_ref, sel_ref, len_ref, out_ref):
    pb = p13_ref[0]
    cg3 = c_ref[0]
    sel = sel_ref[0]
    lb = len_ref[0, 0, 0]
    cu = jnp.dot(sel, cg3, precision=jax.lax.Precision.HIGHEST,
                 preferred_element_type=jnp.float32)
    xb = pb[0:1, :]
    yb = pb[1:2, :]
    zb = pb[2:3, :]
    dx = xb - cu[:, 0:1]
    dy = yb - cu[:, 1:2]
    dz = zb - cu[:, 2:3]
    d2 = dx * dx + dy * dy + dz * dz
    iota_n = jax.lax.broadcasted_iota(jnp.int32, (LU, N), 1)
    d2 = jnp.where(iota_n < lb, d2, jnp.inf)
    groups = []
    for _ in range(K):
        m = jnp.min(d2, axis=1, keepdims=True)
        idx = jnp.min(jnp.where(d2 == m, iota_n, N), axis=1, keepdims=True)
        oh = iota_n == idx
        gx = jnp.sum(jnp.where(oh, xb, 0.0), axis=1, keepdims=True)
        gy = jnp.sum(jnp.where(oh, yb, 0.0), axis=1, keepdims=True)
        gz = jnp.sum(jnp.where(oh, zb, 0.0), axis=1, keepdims=True)
        g = jnp.concatenate([gx, gy, gz], axis=1)
        groups.append((g - cu)[:, None, :])
        d2 = jnp.where(oh, jnp.inf, d2)
    out_ref[0] = jnp.concatenate(groups, axis=1)


def _mlp_kernel(x_ref, w1_ref, b1_ref, w2_ref, b2_ref, w3a_ref, w3b_ref,
                b3_ref, w4_ref, b4_ref, out_ref):
    x = x_ref[...].reshape(GB * K, C)
    f1 = jnp.maximum(jnp.dot(x, w1_ref[...],
                             preferred_element_type=jnp.float32)
                     + b1_ref[...], 0.0)
    f1 = jnp.dot(f1, w2_ref[...],
                 preferred_element_type=jnp.float32) + b2_ref[...]
    gfeat = jnp.max(f1.reshape(GB, K, 256), axis=1)
    h = jnp.dot(f1, w3a_ref[...], preferred_element_type=jnp.float32)
    hg = jnp.dot(gfeat, w3b_ref[...],
                 preferred_element_type=jnp.float32) + b3_ref[...]
    h = h.reshape(GB, K, 512) + hg[:, None, :]
    f3 = jnp.maximum(h, 0.0).reshape(GB * K, 512)
    f4 = jnp.dot(f3, w4_ref[...],
                 preferred_element_type=jnp.float32) + b4_ref[...]
    out_ref[...] = jnp.max(f4.reshape(GB, K, TOKEN_DIM), axis=1)


def kernel(points, lengths, W1, b1, W2, b2, W3, b3, W4, b4):
    perm = jax.vmap(lambda k: jax.random.permutation(k, G))(
        jax.random.split(jax.random.key(42), B)).astype(jnp.int32)
    ui = perm[:, NUM_MASKED:]
    sel = jax.nn.one_hot(ui, G, dtype=jnp.float32)

    p13 = points.transpose(0, 2, 1)
    lens3 = lengths.reshape(B, 1, 1)

    centers13 = pl.pallas_call(
        _fps_kernel,
        out_shape=jax.ShapeDtypeStruct((B, 3, G), jnp.float32),
    )(p13, lens3)
    centers_g3 = centers13.transpose(0, 2, 1)

    groups = pl.pallas_call(
        _knn_kernel,
        grid=(B,),
        in_specs=[
            pl.BlockSpec((1, 3, N), lambda b: (b, 0, 0)),
            pl.BlockSpec((1, G, C), lambda b: (b, 0, 0)),
            pl.BlockSpec((1, LU, G), lambda b: (b, 0, 0)),
            pl.BlockSpec((1, 1, 1), lambda b: (b, 0, 0)),
        ],
        out_specs=pl.BlockSpec((1, LU, K, C), lambda b: (b, 0, 0, 0)),
        out_shape=jax.ShapeDtypeStruct((B, LU, K, C), jnp.float32),
    )(p13, centers_g3, sel, lens3)

    flat_g = groups.reshape(B * LU, K, C)
    nblk = (B * LU) // GB
    tokens = pl.pallas_call(
        _mlp_kernel,
        grid=(nblk,),
        in_specs=[
            pl.BlockSpec((GB, K, C), lambda i: (i, 0, 0)),
            pl.BlockSpec((C, 128), lambda i: (0, 0)),
            pl.BlockSpec((1, 128), lambda i: (0, 0)),
            pl.BlockSpec((128, 256), lambda i: (0, 0)),
            pl.BlockSpec((1, 256), lambda i: (0, 0)),
            pl.BlockSpec((256, 512), lambda i: (0, 0)),
            pl.BlockSpec((256, 512), lambda i: (0, 0)),
            pl.BlockSpec((1, 512), lambda i: (0, 0)),
            pl.BlockSpec((512, TOKEN_DIM), lambda i: (0, 0)),
            pl.BlockSpec((1, TOKEN_DIM), lambda i: (0, 0)),
        ],
        out_specs=pl.BlockSpec((GB, TOKEN_DIM), lambda i: (i, 0)),
        out_shape=jax.ShapeDtypeStruct((B * LU, TOKEN_DIM), jnp.float32),
    )(flat_g, W1, b1.reshape(1, -1), W2, b2.reshape(1, -1),
      W3[:256], W3[256:], b3.reshape(1, -1), W4, b4.reshape(1, -1))

    return tokens.reshape(B, LU, TOKEN_DIM)


# speedup vs baseline: 15.0906x; 1.0829x over previous
import jax
import jax.numpy as jnp
from jax.experimental import pallas as pl
from jax.experimental.pallas import tpu as pltpu

B, N, C = 8, 2048, 3
G = 128
K = 32
TOKEN_DIM = 384
NUM_MASKED = int(G * 0.6)
LU = G - NUM_MASKED
GB = 32


def _fps_kernel(p_ref, len_ref, c_ref):
    x = p_ref[:, 0, :]
    y = p_ref[:, 1, :]
    z = p_ref[:, 2, :]
    lens = len_ref[:, 0, :]
    iota_n = jax.lax.broadcasted_iota(jnp.int32, (B, N), 1)
    valid = iota_n < lens
    iota_g = jax.lax.broadcasted_iota(jnp.int32, (B, G), 1)
    inf = jnp.inf

    dist0 = jnp.where(valid, inf, -inf)
    zeros_g = jnp.zeros((B, G), jnp.float32)
    last0 = jnp.zeros((B, 1), jnp.int32)

    def body(i, carry):
        dist, cxs, cys, czs, last = carry
        ohl = (iota_n == last).astype(jnp.float32)
        cx = jnp.sum(x * ohl, axis=1, keepdims=True)
        cy = jnp.sum(y * ohl, axis=1, keepdims=True)
        cz = jnp.sum(z * ohl, axis=1, keepdims=True)
        upd = iota_g == (i - 1)
        cxs = jnp.where(upd, cx, cxs)
        cys = jnp.where(upd, cy, cys)
        czs = jnp.where(upd, cz, czs)
        dx = x - cx
        dy = y - cy
        dz = z - cz
        d = dx * dx + dy * dy + dz * dz
        dist = jnp.where(valid, jnp.minimum(dist, d), -inf)
        m = jnp.max(dist, axis=1, keepdims=True)
        nxt = jnp.min(jnp.where(dist == m, iota_n, N), axis=1, keepdims=True)
        return dist, cxs, cys, czs, nxt

    _, cxs, cys, czs, last = jax.lax.fori_loop(
        1, G, body, (dist0, zeros_g, zeros_g, zeros_g, last0))
    ohl = (iota_n == last).astype(jnp.float32)
    upd = iota_g == (G - 1)
    cxs = jnp.where(upd, jnp.sum(x * ohl, axis=1, keepdims=True), cxs)
    cys = jnp.where(upd, jnp.sum(y * ohl, axis=1, keepdims=True), cys)
    czs = jnp.where(upd, jnp.sum(z * ohl, axis=1, keepdims=True), czs)
    c_ref[:, 0, :] = cxs
    c_ref[:, 1, :] = cys
    c_ref[:, 2, :] = czs


def _knn_kernel(p13_ref, c13_ref, sel_ref, len_ref, out_ref):
    x = p13_ref[:, 0:1, :]
    y = p13_ref[:, 1:2, :]
    z = p13_ref[:, 2:3, :]
    sel = sel_ref[...]
    cux = jnp.sum(sel * c13_ref[:, 0:1, :], axis=2, keepdims=True)
    cuy = jnp.sum(sel * c13_ref[:, 1:2, :], axis=2, keepdims=True)
    cuz = jnp.sum(sel * c13_ref[:, 2:3, :], axis=2, keepdims=True)
    dx = x - cux
    dy = y - cuy
    dz = z - cuz
    d2 = dx * dx + dy * dy + dz * dz
    iota_n = jax.lax.broadcasted_iota(jnp.int32, (B, LU, N), 2)
    d2 = jnp.where(iota_n < len_ref[...], d2, jnp.inf)
    groups = []
    for _ in range(K):
        m = jnp.min(d2, axis=2, keepdims=True)
        idx = jnp.min(jnp.where(d2 == m, iota_n, N), axis=2, keepdims=True)
        oh = iota_n == idx
        gx = jnp.sum(jnp.where(oh, x, 0.0), axis=2, keepdims=True)
        gy = jnp.sum(jnp.where(oh, y, 0.0), axis=2, keepdims=True)
        gz = jnp.sum(jnp.where(oh, z, 0.0), axis=2, keepdims=True)
        g = jnp.concatenate([gx - cux, gy - cuy, gz - cuz], axis=2)
        groups.append(g[:, :, None, :])
        d2 = jnp.where(oh, jnp.inf, d2)
    out_ref[...] = jnp.concatenate(groups, axis=2)


def _mlp_kernel(x_ref, w1_ref, b1_ref, w2_ref, b2_ref, w3a_ref, w3b_ref,
                b3_ref, w4_ref, b4_ref, out_ref):
    x = x_ref[...].reshape(GB * K, C)
    f1 = jnp.maximum(jnp.dot(x, w1_ref[...],
                             preferred_element_type=jnp.float32)
                     + b1_ref[...], 0.0)
    f1 = jnp.dot(f1, w2_ref[...],
                 preferred_element_type=jnp.float32) + b2_ref[...]
    gfeat = jnp.max(f1.reshape(GB, K, 256), axis=1)
    h = jnp.dot(f1, w3a_ref[...], preferred_element_type=jnp.float32)
    hg = jnp.dot(gfeat, w3b_ref[...],
                 preferred_element_type=jnp.float32) + b3_ref[...]
    h = h.reshape(GB, K, 512) + hg[:, None, :]
    f3 = jnp.maximum(h, 0.0).reshape(GB * K, 512)
    f4 = jnp.dot(f3, w4_ref[...],
                 preferred_element_type=jnp.float32) + b4_ref[...]
    out_ref[...] = jnp.max(f4.reshape(GB, K, TOKEN_DIM), axis=1)


def kernel(points, lengths, W1, b1, W2, b2, W3, b3, W4, b4):
    perm = jax.vmap(lambda k: jax.random.permutation(k, G))(
        jax.random.split(jax.random.key(42), B)).astype(jnp.int32)
    ui = perm[:, NUM_MASKED:]
    sel = jax.nn.one_hot(ui, G, dtype=jnp.float32)

    p13 = points.transpose(0, 2, 1)
    lens3 = lengths.reshape(B, 1, 1)

    centers13 = pl.pallas_call(
        _fps_kernel,
        out_shape=jax.ShapeDtypeStruct((B, 3, G), jnp.float32),
    )(p13, lens3)

    groups = pl.pallas_call(
        _knn_kernel,
        out_shape=jax.ShapeDtypeStruct((B, LU, K, C), jnp.float32),
    )(p13, centers13, sel, lens3)

    flat_g = groups.reshape(B * LU, K, C)
    nblk = (B * LU) // GB
    tokens = pl.pallas_call(
        _mlp_kernel,
        grid=(nblk,),
        in_specs=[
            pl.BlockSpec((GB, K, C), lambda i: (i, 0, 0)),
            pl.BlockSpec((C, 128), lambda i: (0, 0)),
            pl.BlockSpec((1, 128), lambda i: (0, 0)),
            pl.BlockSpec((128, 256), lambda i: (0, 0)),
            pl.BlockSpec((1, 256), lambda i: (0, 0)),
            pl.BlockSpec((256, 512), lambda i: (0, 0)),
            pl.BlockSpec((256, 512), lambda i: (0, 0)),
            pl.BlockSpec((1, 512), lambda i: (0, 0)),
            pl.BlockSpec((512, TOKEN_DIM), lambda i: (0, 0)),
            pl.BlockSpec((1, TOKEN_DIM), lambda i: (0, 0)),
        ],
        out_specs=pl.BlockSpec((GB, TOKEN_DIM), lambda i: (i, 0)),
        out_shape=jax.ShapeDtypeStruct((B * LU, TOKEN_DIM), jnp.float32),
    )(flat_g, W1, b1.reshape(1, -1), W2, b2.reshape(1, -1),
      W3[:256], W3[256:], b3.reshape(1, -1), W4, b4.reshape(1, -1))

    return tokens.reshape(B, LU, TOKEN_DIM)


# speedup vs baseline: 15.6566x; 1.0375x over previous
import jax
import jax.numpy as jnp
from jax.experimental import pallas as pl
from jax.experimental.pallas import tpu as pltpu

B, N, C = 8, 2048, 3
G = 128
K = 32
TOKEN_DIM = 384
NUM_MASKED = int(G * 0.6)
LU = G - NUM_MASKED
GB = 32


def _fps_kernel(p_ref, len_ref, c_ref):
    x = p_ref[:, 0, :]
    y = p_ref[:, 1, :]
    z = p_ref[:, 2, :]
    lens = len_ref[:, 0, :]
    iota_n = jax.lax.broadcasted_iota(jnp.int32, (B, N), 1)
    valid = iota_n < lens
    iota_g = jax.lax.broadcasted_iota(jnp.int32, (B, G), 1)
    inf = jnp.inf

    dist0 = jnp.where(valid, inf, -inf)
    zeros_g = jnp.zeros((B, G), jnp.float32)
    last0 = jnp.zeros((B, 1), jnp.int32)

    def body(i, carry):
        dist, cxs, cys, czs, last = carry
        ohl = (iota_n == last).astype(jnp.float32)
        cx = jnp.sum(x * ohl, axis=1, keepdims=True)
        cy = jnp.sum(y * ohl, axis=1, keepdims=True)
        cz = jnp.sum(z * ohl, axis=1, keepdims=True)
        upd = iota_g == (i - 1)
        cxs = jnp.where(upd, cx, cxs)
        cys = jnp.where(upd, cy, cys)
        czs = jnp.where(upd, cz, czs)
        dx = x - cx
        dy = y - cy
        dz = z - cz
        d = dx * dx + dy * dy + dz * dz
        dist = jnp.where(valid, jnp.minimum(dist, d), -inf)
        nxt = jnp.argmax(dist, axis=1)[:, None].astype(jnp.int32)
        return dist, cxs, cys, czs, nxt

    _, cxs, cys, czs, last = jax.lax.fori_loop(
        1, G, body, (dist0, zeros_g, zeros_g, zeros_g, last0))
    ohl = (iota_n == last).astype(jnp.float32)
    upd = iota_g == (G - 1)
    cxs = jnp.where(upd, jnp.sum(x * ohl, axis=1, keepdims=True), cxs)
    cys = jnp.where(upd, jnp.sum(y * ohl, axis=1, keepdims=True), cys)
    czs = jnp.where(upd, jnp.sum(z * ohl, axis=1, keepdims=True), czs)
    c_ref[:, 0, :] = cxs
    c_ref[:, 1, :] = cys
    c_ref[:, 2, :] = czs


def _knn_kernel(p13_ref, c13_ref, sel_ref, len_ref, out_ref):
    x = p13_ref[:, 0:1, :]
    y = p13_ref[:, 1:2, :]
    z = p13_ref[:, 2:3, :]
    sel = sel_ref[...]
    cux = jnp.sum(sel * c13_ref[:, 0:1, :], axis=2, keepdims=True)
    cuy = jnp.sum(sel * c13_ref[:, 1:2, :], axis=2, keepdims=True)
    cuz = jnp.sum(sel * c13_ref[:, 2:3, :], axis=2, keepdims=True)
    dx = x - cux
    dy = y - cuy
    dz = z - cuz
    d2 = dx * dx + dy * dy + dz * dz
    iota_n = jax.lax.broadcasted_iota(jnp.int32, (B, LU, N), 2)
    d2 = jnp.where(iota_n < len_ref[...], d2, jnp.inf)
    groups = []
    for _ in range(K):
        idx = jnp.argmin(d2, axis=2)[:, :, None].astype(jnp.int32)
        oh = iota_n == idx
        gx = jnp.sum(jnp.where(oh, x, 0.0), axis=2, keepdims=True)
        gy = jnp.sum(jnp.where(oh, y, 0.0), axis=2, keepdims=True)
        gz = jnp.sum(jnp.where(oh, z, 0.0), axis=2, keepdims=True)
        g = jnp.concatenate([gx - cux, gy - cuy, gz - cuz], axis=2)
        groups.append(g[:, :, None, :])
        d2 = jnp.where(oh, jnp.inf, d2)
    out_ref[...] = jnp.concatenate(groups, axis=2)


def _mlp_kernel(x_ref, w1_ref, b1_ref, w2_ref, b2_ref, w3a_ref, w3b_ref,
                b3_ref, w4_ref, b4_ref, out_ref):
    x = x_ref[...].reshape(GB * K, C)
    f1 = jnp.maximum(jnp.dot(x, w1_ref[...],
                             preferred_element_type=jnp.float32)
                     + b1_ref[...], 0.0)
    f1 = jnp.dot(f1, w2_ref[...],
                 preferred_element_type=jnp.float32) + b2_ref[...]
    gfeat = jnp.max(f1.reshape(GB, K, 256), axis=1)
    h = jnp.dot(f1, w3a_ref[...], preferred_element_type=jnp.float32)
    hg = jnp.dot(gfeat, w3b_ref[...],
                 preferred_element_type=jnp.float32) + b3_ref[...]
    h = h.reshape(GB, K, 512) + hg[:, None, :]
    f3 = jnp.maximum(h, 0.0).reshape(GB * K, 512)
    f4 = jnp.dot(f3, w4_ref[...],
                 preferred_element_type=jnp.float32) + b4_ref[...]
    out_ref[...] = jnp.max(f4.reshape(GB, K, TOKEN_DIM), axis=1)


def kernel(points, lengths, W1, b1, W2, b2, W3, b3, W4, b4):
    perm = jax.vmap(lambda k: jax.random.permutation(k, G))(
        jax.random.split(jax.random.key(42), B)).astype(jnp.int32)
    ui = perm[:, NUM_MASKED:]
    sel = jax.nn.one_hot(ui, G, dtype=jnp.float32)

    p13 = points.transpose(0, 2, 1)
    lens3 = lengths.reshape(B, 1, 1)

    centers13 = pl.pallas_call(
        _fps_kernel,
        out_shape=jax.ShapeDtypeStruct((B, 3, G), jnp.float32),
    )(p13, lens3)

    groups = pl.pallas_call(
        _knn_kernel,
        out_shape=jax.ShapeDtypeStruct((B, LU, K, C), jnp.float32),
    )(p13, centers13, sel, lens3)

    flat_g = groups.reshape(B * LU, K, C)
    nblk = (B * LU) // GB
    tokens = pl.pallas_call(
        _mlp_kernel,
        grid=(nblk,),
        in_specs=[
            pl.BlockSpec((GB, K, C), lambda i: (i, 0, 0)),
            pl.BlockSpec((C, 128), lambda i: (0, 0)),
            pl.BlockSpec((1, 128), lambda i: (0, 0)),
            pl.BlockSpec((128, 256), lambda i: (0, 0)),
            pl.BlockSpec((1, 256), lambda i: (0, 0)),
            pl.BlockSpec((256, 512), lambda i: (0, 0)),
            pl.BlockSpec((256, 512), lambda i: (0, 0)),
            pl.BlockSpec((1, 512), lambda i: (0, 0)),
            pl.BlockSpec((512, TOKEN_DIM), lambda i: (0, 0)),
            pl.BlockSpec((1, TOKEN_DIM), lambda i: (0, 0)),
        ],
        out_specs=pl.BlockSpec((GB, TOKEN_DIM), lambda i: (i, 0)),
        out_shape=jax.ShapeDtypeStruct((B * LU, TOKEN_DIM), jnp.float32),
    )(flat_g, W1, b1.reshape(1, -1), W2, b2.reshape(1, -1),
      W3[:256], W3[256:], b3.reshape(1, -1), W4, b4.reshape(1, -1))

    return tokens.reshape(B, LU, TOKEN_DIM)
